# baseline (speedup 1.0000x reference)
import jax
import jax.numpy as jnp
from jax import lax
from jax.experimental import pallas as pl
from jax.experimental.pallas import tpu as pltpu
from jax.experimental.pallas import tpu_sc as plsc

LATENT = 32
NC = 2
NS = 16
NW = NC * NS
L = 16
CHUNK = 128


def _gmf_body(uidx_hbm, iidx_hbm, utab_hbm, itab_hbm, w_hbm, b_hbm, out_hbm,
              uidx_v, iidx_v, urows_v, irows_v, w_v, b_v, sums_v, out_v, sem):
    wid = lax.axis_index("s") * NC + lax.axis_index("c")
    bpw = out_v.shape[0]
    nch = uidx_v.shape[0]
    base = wid * bpw

    pltpu.sync_copy(uidx_hbm.at[wid], uidx_v)
    pltpu.sync_copy(iidx_hbm.at[wid], iidx_v)
    pltpu.sync_copy(w_hbm, w_v)
    pltpu.sync_copy(b_hbm, b_v)

    handles = []
    for j in range(nch):
        handles.append(pltpu.async_copy(
            utab_hbm.at[uidx_v.at[j]], urows_v.at[pl.ds(j * CHUNK, CHUNK)], sem))
        handles.append(pltpu.async_copy(
            itab_hbm.at[iidx_v.at[j]], irows_v.at[pl.ds(j * CHUNK, CHUNK)], sem))
    for h in handles:
        h.wait()

    b_vec = b_v[...]
    w_lo = w_v[pl.ds(0, L)]
    w_hi = w_v[pl.ds(L, L)]
    lanes = lax.iota(jnp.int32, L)

    def group(g, carry):
        rbase = g * L
        for rr in range(L):
            r = rbase + rr
            u0 = urows_v[r, pl.ds(0, L)]
            u1 = urows_v[r, pl.ds(L, L)]
            i0 = irows_v[r, pl.ds(0, L)]
            i1 = irows_v[r, pl.ds(L, L)]
            p = u0 * i0 * w_lo + u1 * i1 * w_hi
            off = pl.multiple_of(r * L, L)
            sums_v[pl.ds(off, L)] = p
        fbase = rbase * L + lanes * L
        acc = b_vec
        for k in range(L):
            acc = acc + plsc.load_gather(sums_v, [fbase + k])
        off = pl.multiple_of(rbase, L)
        out_v[pl.ds(off, L)] = 1.0 / (1.0 + jnp.exp(-acc))
        return carry

    lax.fori_loop(0, bpw // L, group, 0)
    pltpu.sync_copy(out_v, out_hbm.at[pl.ds(base, bpw)])


def kernel(user_indices, item_indices, emb_user_gmf, emb_item_gmf, W_aff, b_aff):
    batch = user_indices.shape[0]
    bpw = batch // NW
    nch = bpw // CHUNK
    uidx = user_indices.astype(jnp.int32).reshape(NW, nch, CHUNK)
    iidx = item_indices.astype(jnp.int32).reshape(NW, nch, CHUNK)
    w = W_aff.reshape(LATENT).astype(jnp.float32)
    b = jnp.broadcast_to(b_aff.reshape(()), (L,)).astype(jnp.float32)

    fn = pl.kernel(
        _gmf_body,
        mesh=plsc.VectorSubcoreMesh(core_axis_name="c", subcore_axis_name="s"),
        compiler_params=pltpu.CompilerParams(
            needs_layout_passes=False, use_tc_tiling_on_sc=False),
        out_type=jax.ShapeDtypeStruct((batch,), jnp.float32),
        scratch_types=[
            pltpu.VMEM((nch, CHUNK), jnp.int32),
            pltpu.VMEM((nch, CHUNK), jnp.int32),
            pltpu.VMEM((bpw, LATENT), jnp.float32),
            pltpu.VMEM((bpw, LATENT), jnp.float32),
            pltpu.VMEM((LATENT,), jnp.float32),
            pltpu.VMEM((L,), jnp.float32),
            pltpu.VMEM((bpw * L,), jnp.float32),
            pltpu.VMEM((bpw,), jnp.float32),
            pltpu.SemaphoreType.DMA,
        ],
    )
    out = fn(uidx, iidx, emb_user_gmf, emb_item_gmf, w, b)
    return out.reshape(batch, 1)
